# baseline (speedup 1.0000x reference)
import functools

import jax
import jax.numpy as jnp
import numpy as np
from jax import lax
from jax.experimental import pallas as pl
from jax.experimental.pallas import tpu as pltpu
from jax.experimental.pallas import tpu_sc as plsc

B, N, D = 16, 576, 768
NTE, NCE, TOPK = 4, 4, 2
E = NTE + NCE
HT = N * 4
HC = D * 4
EPS = 1e-6
S = B * TOPK
J = 3
HT_T = HT // J
HC_T = HC // J


def _gelu(x):
    return 0.5 * x * (1.0 + jnp.tanh(np.sqrt(2.0 / np.pi) * (x + 0.044715 * x ** 3)))


def _router_body(x_ref, rw_ref, lt_ref, aux_ref, ri_ref):
    b = pl.program_id(0)
    ri_ref[b, :] = jnp.mean(x_ref[0], axis=0)

    @pl.when(b == B - 1)
    def _():
        lt_ref[...] = jax.lax.dot_general(
            rw_ref[...], ri_ref[...], (((1,), (1,)), ((), ())),
            preferred_element_type=jnp.float32)
        logits = jax.lax.dot_general(
            ri_ref[...], rw_ref[...], (((1,), (1,)), ((), ())),
            preferred_element_type=jnp.float32)
        m = jnp.max(logits, axis=-1, keepdims=True)
        p = jnp.exp(logits - m)
        p = p / jnp.sum(p, axis=-1, keepdims=True)
        colid = jax.lax.broadcasted_iota(jnp.int32, (B, E), 1)
        v1 = jnp.max(p, axis=-1, keepdims=True)
        i1 = jnp.min(jnp.where(p == v1, colid, E), axis=-1, keepdims=True)
        ep = jnp.mean(p, axis=0, keepdims=True)
        ef = jnp.mean(jnp.where(colid == i1, 1.0, 0.0), axis=0, keepdims=True)
        aux_ref[...] = E * jnp.sum(ep * ef, axis=1, keepdims=True)


def _dispatch_body(sb_ref, sw_ref, se_ref, th_ref, ch_ref,
                   x_ref, tw1_ref, tw2_ref, cw1_ref, cw2_ref,
                   out_ref, xnt_ref, xnc_ref):
    s = pl.program_id(0)
    e = se_ref[s]
    w = sw_ref[s]

    @pl.when(e < NTE)
    def _():
        xx = x_ref[0]
        mt = jnp.mean(xx, axis=0, keepdims=True)
        vt = jnp.mean((xx - mt) ** 2, axis=0, keepdims=True)
        xnt_ref[...] = ((xx - mt) * jax.lax.rsqrt(vt + EPS)).astype(jnp.bfloat16)
        h = jax.lax.dot_general(
            tw1_ref[0], xnt_ref[...], (((1,), (0,)), ((), ())),
            preferred_element_type=jnp.float32)
        g = _gelu(h).astype(jnp.bfloat16)
        y = jax.lax.dot_general(
            tw2_ref[0], g, (((1,), (0,)), ((), ())),
            preferred_element_type=jnp.float32)
        out_ref[...] = (w * y)[None]

    @pl.when(e >= NTE)
    def _():
        xx = x_ref[0]
        mc = jnp.mean(xx, axis=1, keepdims=True)
        vc = jnp.mean((xx - mc) ** 2, axis=1, keepdims=True)
        xnc_ref[...] = ((xx - mc) * jax.lax.rsqrt(vc + EPS)).astype(jnp.bfloat16)
        h = jax.lax.dot_general(
            xnc_ref[...], cw1_ref[0], (((1,), (1,)), ((), ())),
            preferred_element_type=jnp.float32)
        g = _gelu(h).astype(jnp.bfloat16)
        y = jax.lax.dot_general(
            g, cw2_ref[0], (((1,), (1,)), ((), ())),
            preferred_element_type=jnp.float32)
        out_ref[...] = (w * y)[None]


def _router_call(x, router_w):
    return pl.pallas_call(
        _router_body,
        grid=(B,),
        in_specs=[
            pl.BlockSpec((1, N, D), lambda b: (b, 0, 0)),
            pl.BlockSpec((E, D), lambda b: (0, 0)),
        ],
        out_specs=[
            pl.BlockSpec((E, B), lambda b: (0, 0)),
            pl.BlockSpec((1, 1), lambda b: (0, 0)),
        ],
        out_shape=[
            jax.ShapeDtypeStruct((E, B), jnp.float32),
            jax.ShapeDtypeStruct((1, 1), jnp.float32),
        ],
        scratch_shapes=[pltpu.VMEM((B, D), jnp.float32)],
        compiler_params=pltpu.CompilerParams(
            dimension_semantics=("arbitrary",)),
    )(x, router_w)


_SC_MESH = plsc.VectorSubcoreMesh(core_axis_name="c", subcore_axis_name="s")


@functools.partial(
    pl.kernel,
    mesh=_SC_MESH,
    out_type=[
        jax.ShapeDtypeStruct((TOPK, B), jnp.int32),
        jax.ShapeDtypeStruct((TOPK, B), jnp.float32),
    ],
    scratch_types=[
        pltpu.VMEM((E, B), jnp.float32),
        pltpu.VMEM((TOPK, B), jnp.int32),
        pltpu.VMEM((TOPK, B), jnp.float32),
    ],
)
def _sc_router(lt_hbm, idx_hbm, wts_hbm, lt_v, idx_v, wts_v):
    @pl.when((lax.axis_index("c") == 0) & (lax.axis_index("s") == 0))
    def _():
        pltpu.sync_copy(lt_hbm, lt_v)
        rows = [lt_v[e] for e in range(E)]
        m = rows[0]
        for e in range(1, E):
            m = jnp.maximum(m, rows[e])
        ex = [jnp.exp(r - m) for r in rows]
        tot = ex[0]
        for e in range(1, E):
            tot = tot + ex[e]
        p = [v / tot for v in ex]
        zeros = jnp.full((B,), 0.0, jnp.float32)
        neg1 = jnp.full((B,), -1.0, jnp.float32)
        eids = [jnp.full((B,), e, jnp.int32) for e in range(E)]
        v1 = p[0]
        i1 = eids[0]
        for e in range(1, E):
            upd = p[e] > v1
            i1 = jnp.where(upd, eids[e], i1)
            v1 = jnp.where(upd, p[e], v1)
        v2 = neg1
        i2 = eids[0]
        for e in range(E):
            cand = jnp.where(i1 == eids[e], neg1, p[e])
            upd = cand > v2
            i2 = jnp.where(upd, eids[e], i2)
            v2 = jnp.where(upd, cand, v2)
        s = v1 + v2
        idx_v[0] = i1
        idx_v[1] = i2
        wts_v[0] = v1 / s
        wts_v[1] = v2 / s
        pltpu.sync_copy(idx_v, idx_hbm)
        pltpu.sync_copy(wts_v, wts_hbm)


def _hold_prev(se_flat, is_mine, val):
    pos = jnp.arange(se_flat.shape[0], dtype=jnp.int32)
    ff = jax.lax.cummax(jnp.where(is_mine, pos, -1))
    return jnp.where(ff >= 0, jnp.take(jnp.where(is_mine, val, 0),
                                       jnp.maximum(ff, 0)), 0).astype(jnp.int32)


def _dispatch_call(x, sb, sw, se, tw1, tw2, cw1, cw2):
    th = _hold_prev(se, se < NTE, se)
    ch = _hold_prev(se, se >= NTE, se - NTE)
    grid_spec = pltpu.PrefetchScalarGridSpec(
        num_scalar_prefetch=5,
        grid=(S,),
        in_specs=[
            pl.BlockSpec((1, N, D), lambda s, sb, sw, se, th, ch: (sb[s], 0, 0)),
            pl.BlockSpec((1, HT, N), lambda s, sb, sw, se, th, ch: (th[s], 0, 0)),
            pl.BlockSpec((1, N, HT), lambda s, sb, sw, se, th, ch: (th[s], 0, 0)),
            pl.BlockSpec((1, HC, D), lambda s, sb, sw, se, th, ch: (ch[s], 0, 0)),
            pl.BlockSpec((1, D, HC), lambda s, sb, sw, se, th, ch: (ch[s], 0, 0)),
        ],
        out_specs=pl.BlockSpec((1, N, D), lambda s, sb, sw, se, th, ch: (s, 0, 0)),
        scratch_shapes=[
            pltpu.VMEM((N, D), jnp.bfloat16),
            pltpu.VMEM((N, D), jnp.bfloat16),
        ],
    )
    return pl.pallas_call(
        _dispatch_body,
        grid_spec=grid_spec,
        out_shape=jax.ShapeDtypeStruct((S, N, D), jnp.float32),
        compiler_params=pltpu.CompilerParams(
            dimension_semantics=("arbitrary",)),
    )(sb, sw, se, th, ch, x, tw1, tw2, cw1, cw2)


def _combine_body(s0_ref, s1_ref, a_ref, b_ref, out_ref):
    out_ref[...] = a_ref[...] + b_ref[...]


def _combine_call(slot_out, s0, s1):
    grid_spec = pltpu.PrefetchScalarGridSpec(
        num_scalar_prefetch=2,
        grid=(B,),
        in_specs=[
            pl.BlockSpec((1, N, D), lambda b, s0, s1: (s0[b], 0, 0)),
            pl.BlockSpec((1, N, D), lambda b, s0, s1: (s1[b], 0, 0)),
        ],
        out_specs=pl.BlockSpec((1, N, D), lambda b, s0, s1: (b, 0, 0)),
    )
    return pl.pallas_call(
        _combine_body,
        grid_spec=grid_spec,
        out_shape=jax.ShapeDtypeStruct((B, N, D), jnp.float32),
        compiler_params=pltpu.CompilerParams(
            dimension_semantics=("arbitrary",)),
    )(s0, s1, slot_out, slot_out)


def kernel(x, router_w, tm_fc1_w, tm_fc1_b, tm_fc2_w, tm_fc2_b,
           cm_fc1_w, cm_fc1_b, cm_fc2_w, cm_fc2_b):
    logits_t, aux = _router_call(x, router_w)
    idx_t, wts_t = _sc_router(logits_t)
    sexp = idx_t.T
    swts = wts_t.T
    se_flat = sexp.reshape(-1)
    order = jnp.argsort(se_flat, stable=True).astype(jnp.int32)
    sb = (order // TOPK).astype(jnp.int32)
    se = jnp.take(se_flat, order)
    sw = jnp.take(swts.reshape(-1), order)
    inv = jnp.argsort(order, stable=True).astype(jnp.int32)
    s0 = inv[0::TOPK]
    s1 = inv[1::TOPK]
    slot_out = _dispatch_call(x, sb, sw, se,
                              tm_fc1_w.astype(jnp.bfloat16),
                              tm_fc2_w.astype(jnp.bfloat16),
                              cm_fc1_w.astype(jnp.bfloat16),
                              cm_fc2_w.astype(jnp.bfloat16))
    out = _combine_call(slot_out, s0, s1)
    return out, aux[0, 0]


# speedup vs baseline: 1.0799x; 1.0799x over previous
import functools

import jax
import jax.numpy as jnp
import numpy as np
from jax import lax
from jax.experimental import pallas as pl
from jax.experimental.pallas import tpu as pltpu
from jax.experimental.pallas import tpu_sc as plsc

B, N, D = 16, 576, 768
NTE, NCE, TOPK = 4, 4, 2
E = NTE + NCE
HT = N * 4
HC = D * 4
EPS = 1e-6
J = 3
HT_T = HT // J
HC_T = HC // J


def _gelu(x):
    return 0.5 * x * (1.0 + jnp.tanh(np.sqrt(2.0 / np.pi) * (x + 0.044715 * x ** 3)))


def _router_body(x_ref, rw_ref, lt_ref, aux_ref, ri_ref):
    b = pl.program_id(0)
    ri_ref[b, :] = jnp.mean(x_ref[0], axis=0)

    @pl.when(b == B - 1)
    def _():
        lt_ref[...] = jax.lax.dot_general(
            rw_ref[...], ri_ref[...], (((1,), (1,)), ((), ())),
            preferred_element_type=jnp.float32)
        logits = jax.lax.dot_general(
            ri_ref[...], rw_ref[...], (((1,), (1,)), ((), ())),
            preferred_element_type=jnp.float32)
        m = jnp.max(logits, axis=-1, keepdims=True)
        p = jnp.exp(logits - m)
        p = p / jnp.sum(p, axis=-1, keepdims=True)
        colid = jax.lax.broadcasted_iota(jnp.int32, (B, E), 1)
        v1 = jnp.max(p, axis=-1, keepdims=True)
        i1 = jnp.min(jnp.where(p == v1, colid, E), axis=-1, keepdims=True)
        ep = jnp.mean(p, axis=0, keepdims=True)
        ef = jnp.mean(jnp.where(colid == i1, 1.0, 0.0), axis=0, keepdims=True)
        aux_ref[...] = E * jnp.sum(ep * ef, axis=1, keepdims=True)


def _dispatch_body(sexp_ref, swts_ref, th_ref, ch_ref,
                   x_ref, tw1_ref, tw2_ref, cw1_ref, cw2_ref,
                   out_ref, xnt_ref, xnc_ref):
    b, k = pl.program_id(0), pl.program_id(1)
    e = sexp_ref[b, k]
    w = swts_ref[b, k]

    @pl.when(k == 0)
    def _():
        xx = x_ref[0]
        mt = jnp.mean(xx, axis=0, keepdims=True)
        vt = jnp.mean((xx - mt) ** 2, axis=0, keepdims=True)
        xnt_ref[...] = ((xx - mt) * jax.lax.rsqrt(vt + EPS)).astype(jnp.bfloat16)
        mc = jnp.mean(xx, axis=1, keepdims=True)
        vc = jnp.mean((xx - mc) ** 2, axis=1, keepdims=True)
        xnc_ref[...] = ((xx - mc) * jax.lax.rsqrt(vc + EPS)).astype(jnp.bfloat16)

    @pl.when(e < NTE)
    def _():
        h = jax.lax.dot_general(
            tw1_ref[0], xnt_ref[...], (((1,), (0,)), ((), ())),
            preferred_element_type=jnp.float32)
        g = _gelu(h).astype(jnp.bfloat16)
        y = jax.lax.dot_general(
            tw2_ref[0], g, (((1,), (0,)), ((), ())),
            preferred_element_type=jnp.float32)
        contrib = w * y

        @pl.when(k == 0)
        def _():
            out_ref[...] = contrib[None]

        @pl.when(k != 0)
        def _():
            out_ref[...] += contrib[None]

    @pl.when(e >= NTE)
    def _():
        h = jax.lax.dot_general(
            xnc_ref[...], cw1_ref[0], (((1,), (1,)), ((), ())),
            preferred_element_type=jnp.float32)
        g = _gelu(h).astype(jnp.bfloat16)
        y = jax.lax.dot_general(
            g, cw2_ref[0], (((1,), (1,)), ((), ())),
            preferred_element_type=jnp.float32)
        contrib = w * y

        @pl.when(k == 0)
        def _():
            out_ref[...] = contrib[None]

        @pl.when(k != 0)
        def _():
            out_ref[...] += contrib[None]


def _router_call(x, router_w):
    return pl.pallas_call(
        _router_body,
        grid=(B,),
        in_specs=[
            pl.BlockSpec((1, N, D), lambda b: (b, 0, 0)),
            pl.BlockSpec((E, D), lambda b: (0, 0)),
        ],
        out_specs=[
            pl.BlockSpec((E, B), lambda b: (0, 0)),
            pl.BlockSpec((1, 1), lambda b: (0, 0)),
        ],
        out_shape=[
            jax.ShapeDtypeStruct((E, B), jnp.float32),
            jax.ShapeDtypeStruct((1, 1), jnp.float32),
        ],
        scratch_shapes=[pltpu.VMEM((B, D), jnp.float32)],
        compiler_params=pltpu.CompilerParams(
            dimension_semantics=("arbitrary",)),
    )(x, router_w)


_SC_MESH = plsc.VectorSubcoreMesh(core_axis_name="c", subcore_axis_name="s")


@functools.partial(
    pl.kernel,
    mesh=_SC_MESH,
    out_type=[
        jax.ShapeDtypeStruct((TOPK, B), jnp.int32),
        jax.ShapeDtypeStruct((TOPK, B), jnp.float32),
    ],
    scratch_types=[
        pltpu.VMEM((E, B), jnp.float32),
        pltpu.VMEM((TOPK, B), jnp.int32),
        pltpu.VMEM((TOPK, B), jnp.float32),
    ],
)
def _sc_router(lt_hbm, idx_hbm, wts_hbm, lt_v, idx_v, wts_v):
    @pl.when((lax.axis_index("c") == 0) & (lax.axis_index("s") == 0))
    def _():
        pltpu.sync_copy(lt_hbm, lt_v)
        rows = [lt_v[e] for e in range(E)]
        m = rows[0]
        for e in range(1, E):
            m = jnp.maximum(m, rows[e])
        ex = [jnp.exp(r - m) for r in rows]
        tot = ex[0]
        for e in range(1, E):
            tot = tot + ex[e]
        p = [v / tot for v in ex]
        zeros = jnp.full((B,), 0.0, jnp.float32)
        neg1 = jnp.full((B,), -1.0, jnp.float32)
        eids = [jnp.full((B,), e, jnp.int32) for e in range(E)]
        v1 = p[0]
        i1 = eids[0]
        for e in range(1, E):
            upd = p[e] > v1
            i1 = jnp.where(upd, eids[e], i1)
            v1 = jnp.where(upd, p[e], v1)
        v2 = neg1
        i2 = eids[0]
        for e in range(E):
            cand = jnp.where(i1 == eids[e], neg1, p[e])
            upd = cand > v2
            i2 = jnp.where(upd, eids[e], i2)
            v2 = jnp.where(upd, cand, v2)
        s = v1 + v2
        idx_v[0] = i1
        idx_v[1] = i2
        wts_v[0] = v1 / s
        wts_v[1] = v2 / s
        pltpu.sync_copy(idx_v, idx_hbm)
        pltpu.sync_copy(wts_v, wts_hbm)


def _hold_prev(se_flat, is_mine, val):
    pos = jnp.arange(se_flat.shape[0], dtype=jnp.int32)
    ff = jax.lax.cummax(jnp.where(is_mine, pos, -1))
    return jnp.where(ff >= 0, jnp.take(jnp.where(is_mine, val, 0),
                                       jnp.maximum(ff, 0)), 0).astype(jnp.int32)


def _dispatch_call(x, sexp, swts, tw1, tw2, cw1, cw2):
    se_flat = sexp.reshape(-1)
    th = _hold_prev(se_flat, se_flat < NTE, se_flat).reshape(B, TOPK)
    ch = _hold_prev(se_flat, se_flat >= NTE, se_flat - NTE).reshape(B, TOPK)
    grid_spec = pltpu.PrefetchScalarGridSpec(
        num_scalar_prefetch=4,
        grid=(B, TOPK),
        in_specs=[
            pl.BlockSpec((1, N, D), lambda b, k, se, sw, th, ch: (b, 0, 0)),
            pl.BlockSpec((1, HT, N), lambda b, k, se, sw, th, ch: (th[b, k], 0, 0)),
            pl.BlockSpec((1, N, HT), lambda b, k, se, sw, th, ch: (th[b, k], 0, 0)),
            pl.BlockSpec((1, HC, D), lambda b, k, se, sw, th, ch: (ch[b, k], 0, 0)),
            pl.BlockSpec((1, D, HC), lambda b, k, se, sw, th, ch: (ch[b, k], 0, 0)),
        ],
        out_specs=pl.BlockSpec((1, N, D), lambda b, k, se, sw, th, ch: (b, 0, 0)),
        scratch_shapes=[
            pltpu.VMEM((N, D), jnp.bfloat16),
            pltpu.VMEM((N, D), jnp.bfloat16),
        ],
    )
    return pl.pallas_call(
        _dispatch_body,
        grid_spec=grid_spec,
        out_shape=jax.ShapeDtypeStruct((B, N, D), jnp.float32),
        compiler_params=pltpu.CompilerParams(
            dimension_semantics=("arbitrary", "arbitrary")),
    )(sexp, swts, th, ch, x, tw1, tw2, cw1, cw2)


def kernel(x, router_w, tm_fc1_w, tm_fc1_b, tm_fc2_w, tm_fc2_b,
           cm_fc1_w, cm_fc1_b, cm_fc2_w, cm_fc2_b):
    logits_t, aux = _router_call(x, router_w)
    idx_t, wts_t = _sc_router(logits_t)
    sexp = idx_t.T
    swts = wts_t.T
    out = _dispatch_call(x, sexp, swts,
                         tm_fc1_w.astype(jnp.bfloat16),
                         tm_fc2_w.astype(jnp.bfloat16),
                         cm_fc1_w.astype(jnp.bfloat16),
                         cm_fc2_w.astype(jnp.bfloat16))
    return out, aux[0, 0]
